# baseline (speedup 1.0000x reference)
import functools

import jax
import jax.numpy as jnp
from jax import lax
from jax.experimental import pallas as pl
from jax.experimental.pallas import tpu as pltpu
from jax.experimental.pallas import tpu_sc as plsc

_info = plsc.get_sparse_core_info()
_NC = _info.num_cores
_NS = _info.num_subcores
_NW = _NC * _NS
_L = _info.num_lanes

_BLK = 4
_CIDX = _BLK * 128


@functools.lru_cache(maxsize=None)
def _make_gather(S, NB, D, blocks_per_w, n_chunks):
  mesh = plsc.VectorSubcoreMesh(core_axis_name="c", subcore_axis_name="s")
  DT = D // 8

  @functools.partial(
      pl.kernel,
      mesh=mesh,
      out_type=jax.ShapeDtypeStruct((S, DT, NB, 8, 128), jnp.float32),
      scratch_types=[
          pltpu.VMEM((2, _CIDX), jnp.int32),
          pltpu.VMEM((2, _CIDX, D), jnp.float32),
          pltpu.VMEM((2, DT, _BLK, 8, 128), jnp.float32),
          pltpu.SemaphoreType.DMA,
          pltpu.SemaphoreType.DMA,
          pltpu.SemaphoreType.DMA,
      ],
      compiler_params=pltpu.CompilerParams(
          use_tc_tiling_on_sc=False, needs_layout_passes=False),
  )
  def k(table_hbm, idx_hbm, out_hbm, idx_v, rows_v, tile_v, gsem0, gsem1,
        osem):
    wid = lax.axis_index("s") * _NC + lax.axis_index("c")
    blk_base = wid * blocks_per_w
    iota = lax.iota(jnp.int32, _L)
    gsems = (gsem0, gsem1)

    def stage(k_idx, p):
      off = (blk_base + _BLK * k_idx) * 128
      pltpu.sync_copy(idx_hbm.at[pl.ds(off, _CIDX)], idx_v.at[p])
      return pltpu.async_copy(table_hbm.at[idx_v.at[p]], rows_v.at[p],
                              gsems[p])

    def process(k_idx, p):
      pltpu.make_async_copy(table_hbm.at[idx_v.at[p]], rows_v.at[p],
                            gsems[p]).wait()
      nxt = k_idx + 1

      @pl.when(nxt < n_chunks)
      def _():
        stage(nxt, 1 - p)

      blk0 = blk_base + _BLK * k_idx
      s_pos = blk0 // NB
      tj0 = lax.rem(blk0, NB)

      for tjq in range(_BLK):
        row0 = tjq * 128
        for ti in range(DT):
          for r in range(8):
            dvec = jnp.full((_L,), 8 * ti + r, dtype=jnp.int32)
            for c0 in range(0, 128, _L):
              rvec = (row0 + c0) + iota
              v = plsc.load_gather(rows_v.at[p], [rvec, dvec])
              tile_v[p, ti, tjq, r, pl.ds(c0, _L)] = v

      for ti in range(DT):
        pltpu.async_copy(tile_v.at[p, ti],
                         out_hbm.at[s_pos, ti, pl.ds(tj0, _BLK)], osem)
      for ti in range(DT):
        pltpu.make_async_copy(tile_v.at[p, ti],
                              out_hbm.at[s_pos, ti, pl.ds(tj0, _BLK)],
                              osem).wait()

    stage(0, 0)

    def body(kk, carry):
      process(2 * kk, 0)
      process(2 * kk + 1, 1)
      return carry

    lax.fori_loop(0, n_chunks // 2, body, 0)

  return k


def kernel(table, x):
  B0, S = x.shape
  V, D = table.shape
  NB = B0 // 128
  n_blocks = S * NB
  blocks_per_w = n_blocks // _NW
  n_chunks = blocks_per_w // _BLK
  xt = jnp.swapaxes(x, 0, 1).reshape(B0 * S)
  out5 = _make_gather(S, NB, D, blocks_per_w, n_chunks)(table, xt)
  return out5.transpose(2, 4, 0, 1, 3).reshape(B0, S, D)


# speedup vs baseline: 1.3377x; 1.3377x over previous
import functools

import jax
import jax.numpy as jnp
from jax import lax
from jax.experimental import pallas as pl
from jax.experimental.pallas import tpu as pltpu
from jax.experimental.pallas import tpu_sc as plsc

_info = plsc.get_sparse_core_info()
_NC = _info.num_cores
_NS = _info.num_subcores
_NW = _NC * _NS
_L = _info.num_lanes

_BLK = 4
_CIDX = _BLK * 128


@functools.lru_cache(maxsize=None)
def _make_gather(S, NB, D, blocks_per_w, n_chunks):
  mesh = plsc.VectorSubcoreMesh(core_axis_name="c", subcore_axis_name="s")
  DT = D // 8

  @functools.partial(
      pl.kernel,
      mesh=mesh,
      out_type=jax.ShapeDtypeStruct((S, DT, NB, 8, 128), jnp.float32),
      scratch_types=[
          pltpu.VMEM((2, _CIDX), jnp.int32),
          pltpu.VMEM((2, _CIDX, D), jnp.float32),
          pltpu.VMEM((2, DT, _BLK, 8, 128), jnp.float32),
          pltpu.SemaphoreType.DMA,
          pltpu.SemaphoreType.DMA,
          pltpu.SemaphoreType.DMA,
      ],
      compiler_params=pltpu.CompilerParams(
          use_tc_tiling_on_sc=False, needs_layout_passes=False),
  )
  def k(table_hbm, idx_hbm, out_hbm, idx_v, rows_v, tile_v, gsem0, gsem1,
        osem):
    wid = lax.axis_index("s") * _NC + lax.axis_index("c")
    blk_base = wid * blocks_per_w
    iota = lax.iota(jnp.int32, _L)
    gsems = (gsem0, gsem1)

    def stage(k_idx, p):
      off = (blk_base + _BLK * k_idx) * 128
      pltpu.sync_copy(idx_hbm.at[pl.ds(off, _CIDX)], idx_v.at[p])
      return pltpu.async_copy(table_hbm.at[idx_v.at[p]], rows_v.at[p],
                              gsems[p])

    def process(k_idx, p):
      pltpu.make_async_copy(table_hbm.at[idx_v.at[p]], rows_v.at[p],
                            gsems[p]).wait()
      nxt = k_idx + 1

      @pl.when(nxt < n_chunks)
      def _():
        stage(nxt, 1 - p)

      blk0 = blk_base + _BLK * k_idx
      s_pos = blk0 // NB
      tj0 = lax.rem(blk0, NB)

      @plsc.parallel_loop(0, DT * _BLK * 8 * 8, unroll=16)
      def _(m):
        ti = m >> 8
        tjq = (m >> 6) & (_BLK - 1)
        r = (m >> 3) & 7
        c0 = (m & 7) * _L
        dvec = jnp.full((_L,), 8 * ti + r, dtype=jnp.int32)
        rvec = tjq * 128 + c0 + iota
        v = plsc.load_gather(rows_v.at[p], [rvec, dvec])
        tile_v[p, ti, tjq, r, pl.ds(c0, _L)] = v

      for ti in range(DT):
        pltpu.async_copy(tile_v.at[p, ti],
                         out_hbm.at[s_pos, ti, pl.ds(tj0, _BLK)], osem)
      for ti in range(DT):
        pltpu.make_async_copy(tile_v.at[p, ti],
                              out_hbm.at[s_pos, ti, pl.ds(tj0, _BLK)],
                              osem).wait()

    stage(0, 0)

    def body(kk, carry):
      process(2 * kk, 0)
      process(2 * kk + 1, 1)
      return carry

    lax.fori_loop(0, n_chunks // 2, body, 0)

  return k


def kernel(table, x):
  B0, S = x.shape
  V, D = table.shape
  NB = B0 // 128
  n_blocks = S * NB
  blocks_per_w = n_blocks // _NW
  n_chunks = blocks_per_w // _BLK
  xt = jnp.swapaxes(x, 0, 1).reshape(B0 * S)
  out5 = _make_gather(S, NB, D, blocks_per_w, n_chunks)(table, xt)
  return out5.transpose(2, 4, 0, 1, 3).reshape(B0, S, D)


# speedup vs baseline: 1.9472x; 1.4557x over previous
import functools

import jax
import jax.numpy as jnp
from jax import lax
from jax.experimental import pallas as pl
from jax.experimental.pallas import tpu as pltpu
from jax.experimental.pallas import tpu_sc as plsc

_info = plsc.get_sparse_core_info()
_NC = _info.num_cores
_NS = _info.num_subcores
_NW = _NC * _NS
_L = _info.num_lanes

_BLK = 4
_CIDX = _BLK * 128


@functools.lru_cache(maxsize=None)
def _make_gather(S, NB, D, blocks_per_w, n_chunks):
  mesh = plsc.VectorSubcoreMesh(core_axis_name="c", subcore_axis_name="s")
  DT = D // 8
  TSZ = _BLK * 8 * 128

  @functools.partial(
      pl.kernel,
      mesh=mesh,
      out_type=jax.ShapeDtypeStruct((S, DT, NB * 8 * 128), jnp.float32),
      scratch_types=[
          pltpu.VMEM((2, _CIDX), jnp.int32),
          pltpu.VMEM((2, _CIDX, D), jnp.float32),
          pltpu.VMEM((2, DT * TSZ), jnp.float32),
          pltpu.SemaphoreType.DMA,
          pltpu.SemaphoreType.DMA,
          pltpu.SemaphoreType.DMA,
          pltpu.SemaphoreType.DMA,
      ],
      compiler_params=pltpu.CompilerParams(
          use_tc_tiling_on_sc=False, needs_layout_passes=False),
  )
  def k(table_hbm, idx_hbm, out_hbm, idx_v, rows_v, tile_v, gsem0, gsem1,
        osem0, osem1):
    wid = lax.axis_index("s") * _NC + lax.axis_index("c")
    blk_base = wid * blocks_per_w
    iota = lax.iota(jnp.int32, _L)
    gsems = (gsem0, gsem1)
    osems = (osem0, osem1)

    def drain_out(p):
      for ti in range(DT):
        pltpu.make_async_copy(tile_v.at[p, pl.ds(ti * TSZ, TSZ)],
                              out_hbm.at[0, ti, pl.ds(0, TSZ)],
                              osems[p]).wait()

    def stage(k_idx, p):
      off = (blk_base + _BLK * k_idx) * 128
      pltpu.sync_copy(idx_hbm.at[pl.ds(off, _CIDX)], idx_v.at[p])
      return pltpu.async_copy(table_hbm.at[idx_v.at[p]], rows_v.at[p],
                              gsems[p])

    def process(k_idx, p):
      pltpu.make_async_copy(table_hbm.at[idx_v.at[p]], rows_v.at[p],
                            gsems[p]).wait()
      nxt = k_idx + 1

      @pl.when(nxt < n_chunks)
      def _():
        stage(nxt, 1 - p)

      @pl.when(k_idx >= 2)
      def _():
        drain_out(p)

      blk0 = blk_base + _BLK * k_idx
      s_pos = blk0 // NB
      tj0 = lax.rem(blk0, NB)

      @plsc.parallel_loop(0, _BLK * 2 * 8 * _L, unroll=16)
      def _(n):
        m = n & (_L - 1)
        dh = (n >> 4) & 1
        c0 = ((n >> 5) & 7) * _L
        tjq = n >> 8
        dvec = ((iota + m) & (_L - 1)) + dh * _L
        rvec = (tjq * 128 + c0) + iota
        v = plsc.load_gather(rows_v.at[p], [rvec, dvec])
        addr = ((dvec >> 3) * TSZ + ((dvec & 7) * 128)
                + (tjq * 1024 + c0) + iota)
        plsc.store_scatter(tile_v.at[p], [addr], v)

      for ti in range(DT):
        pltpu.async_copy(tile_v.at[p, pl.ds(ti * TSZ, TSZ)],
                         out_hbm.at[s_pos, ti, pl.ds(tj0 * 1024, TSZ)],
                         osems[p])

    stage(0, 0)

    def body(kk, carry):
      process(2 * kk, 0)
      process(2 * kk + 1, 1)
      return carry

    lax.fori_loop(0, n_chunks // 2, body, 0)
    drain_out(0)
    drain_out(1)

  return k


def kernel(table, x):
  B0, S = x.shape
  V, D = table.shape
  NB = B0 // 128
  n_blocks = S * NB
  blocks_per_w = n_blocks // _NW
  n_chunks = blocks_per_w // _BLK
  xt = jnp.swapaxes(x, 0, 1).reshape(B0 * S)
  out5 = _make_gather(S, NB, D, blocks_per_w, n_chunks)(table, xt)
  return (out5.reshape(S, D // 8, NB, 8, 128)
          .transpose(2, 4, 0, 1, 3).reshape(B0, S, D))
